# Initial kernel scaffold; baseline (speedup 1.0000x reference)
import jax
import jax.numpy as jnp
from jax.experimental import pallas as pl


def kernel(not_infected_probs):
    raise NotImplementedError("write your pallas kernel here")


# speedup vs baseline: 108.6772x; 108.6772x over previous
import functools

import numpy as np
import jax
import jax.numpy as jnp
from jax import lax
from jax.experimental import pallas as pl
from jax.experimental.pallas import tpu as pltpu

_LANES = 128

_K0 = np.uint32(908003072)
_K1 = np.uint32(3252900185)
_K2 = np.uint32(_K0 ^ _K1 ^ np.uint32(0x1BD11BDA))
_KS = (_K0, _K1, _K2)
_ROT = ((13, 15, 26, 6), (17, 29, 16, 24))


def _threefry_bits(x1v):
    x0 = jnp.full_like(x1v, _K0)
    x1 = x1v + _K1
    for i in range(5):
        for r in _ROT[i % 2]:
            x0 = x0 + x1
            x1 = (x1 << r) | (x1 >> (32 - r))
            x1 = x0 ^ x1
        x0 = x0 + _KS[(i + 1) % 3]
        x1 = x1 + _KS[(i + 2) % 3] + np.uint32(i + 1)
    return x0 ^ x1


def _neg_log_unit(bits):
    fb = (bits >> 9) | np.uint32(0x3F800000)
    f = lax.bitcast_convert_type(fb, jnp.float32) - np.float32(1.0)
    span = np.float32(np.float32(1.0) - np.float32(1e-10))
    u = jnp.maximum(np.float32(1e-10), f * span + np.float32(1e-10))
    return -jnp.log(u)


def _body(p_ref, o_ref, *, blk_rows, n_elems):
    g = pl.program_id(0)
    p = p_ref[...]
    row = lax.broadcasted_iota(jnp.int32, (blk_rows, _LANES), 0)
    col = lax.broadcasted_iota(jnp.int32, (blk_rows, _LANES), 1)
    base = g * (blk_rows * _LANES)
    idx = (base + row * _LANES + col).astype(jnp.uint32)
    e0 = _neg_log_unit(_threefry_bits(idx))
    e1 = _neg_log_unit(_threefry_bits(idx + np.uint32(n_elems)))
    cond = (np.float32(1.0) - p) * e0 > p * e1
    o_ref[...] = cond.astype(jnp.float32)


def kernel(not_infected_probs):
    n = not_infected_probs.shape[0]
    pad = (-n) % _LANES
    p = not_infected_probs
    if pad:
        p = jnp.pad(p, (0, pad))
    rows = p.shape[0] // _LANES
    p2 = p.reshape(rows, _LANES)

    blk_rows = 256
    grid = (rows + blk_rows - 1) // blk_rows
    out = pl.pallas_call(
        functools.partial(_body, blk_rows=blk_rows, n_elems=n),
        grid=(grid,),
        in_specs=[pl.BlockSpec((blk_rows, _LANES), lambda g: (g, 0))],
        out_specs=pl.BlockSpec((blk_rows, _LANES), lambda g: (g, 0)),
        out_shape=jax.ShapeDtypeStruct((rows, _LANES), jnp.float32),
        compiler_params=pltpu.CompilerParams(
            dimension_semantics=("parallel",),
        ),
    )(p2)
    out = out.reshape(rows * _LANES)
    if pad:
        out = out[:n]
    return out


# speedup vs baseline: 112.3790x; 1.0341x over previous
import functools

import numpy as np
import jax
import jax.numpy as jnp
from jax import lax
from jax.experimental import pallas as pl
from jax.experimental.pallas import tpu as pltpu

_LANES = 128

_K0 = np.uint32(908003072)
_K1 = np.uint32(3252900185)
_K2 = np.uint32(_K0 ^ _K1 ^ np.uint32(0x1BD11BDA))
_KS = (_K0, _K1, _K2)
_ROT = ((13, 15, 26, 6), (17, 29, 16, 24))


def _threefry_bits(x1v):
    x0 = jnp.full_like(x1v, _K0)
    x1 = x1v + _K1
    for i in range(5):
        for r in _ROT[i % 2]:
            x0 = x0 + x1
            x1 = (x1 << r) | (x1 >> (32 - r))
            x1 = x0 ^ x1
        x0 = x0 + _KS[(i + 1) % 3]
        x1 = x1 + np.uint32(_KS[(i + 2) % 3] + np.uint32(i + 1))
    return x0 ^ x1


def _neg_log_unit(bits):
    fb = (bits >> 9) | np.uint32(0x3F800000)
    f = lax.bitcast_convert_type(fb, jnp.float32) - np.float32(1.0)
    span = np.float32(np.float32(1.0) - np.float32(1e-10))
    u = jnp.maximum(np.float32(1e-10), f * span + np.float32(1e-10))
    return -jnp.log(u)


def _body(p_ref, o_ref, *, blk_rows, n_elems):
    g = pl.program_id(0)
    p = p_ref[...]
    row = lax.broadcasted_iota(jnp.int32, (blk_rows, _LANES), 0)
    col = lax.broadcasted_iota(jnp.int32, (blk_rows, _LANES), 1)
    base = g * (blk_rows * _LANES)
    idx = (base + row * _LANES + col).astype(jnp.uint32)
    e0 = _neg_log_unit(_threefry_bits(idx))
    e1 = _neg_log_unit(_threefry_bits(idx + np.uint32(n_elems)))
    cond = (np.float32(1.0) - p) * e0 > p * e1
    o_ref[...] = cond.astype(jnp.float32)


def kernel(not_infected_probs):
    n = not_infected_probs.shape[0]
    pad = (-n) % _LANES
    p = not_infected_probs
    if pad:
        p = jnp.pad(p, (0, pad))
    rows = p.shape[0] // _LANES
    p2 = p.reshape(rows, _LANES)

    blk_rows = 256
    grid = (rows + blk_rows - 1) // blk_rows
    out = pl.pallas_call(
        functools.partial(_body, blk_rows=blk_rows, n_elems=n),
        grid=(grid,),
        in_specs=[pl.BlockSpec((blk_rows, _LANES), lambda g: (g, 0))],
        out_specs=pl.BlockSpec((blk_rows, _LANES), lambda g: (g, 0)),
        out_shape=jax.ShapeDtypeStruct((rows, _LANES), jnp.float32),
        compiler_params=pltpu.CompilerParams(
            dimension_semantics=("parallel",),
        ),
    )(p2)
    out = out.reshape(rows * _LANES)
    if pad:
        out = out[:n]
    return out


# speedup vs baseline: 115.8179x; 1.0306x over previous
import functools

import numpy as np
import jax
import jax.numpy as jnp
from jax import lax
from jax.experimental import pallas as pl
from jax.experimental.pallas import tpu as pltpu

_LANES = 128

_K0 = np.uint32(908003072)
_K1 = np.uint32(3252900185)
_K2 = np.uint32(_K0 ^ _K1 ^ np.uint32(0x1BD11BDA))
_KS = (_K0, _K1, _K2)
_ROT = ((13, 15, 26, 6), (17, 29, 16, 24))


def _threefry_bits(x1v):
    x0 = jnp.full_like(x1v, _K0)
    x1 = x1v + _K1
    for i in range(5):
        for r in _ROT[i % 2]:
            x0 = x0 + x1
            x1 = (x1 << r) | (x1 >> (32 - r))
            x1 = x0 ^ x1
        x0 = x0 + _KS[(i + 1) % 3]
        x1 = x1 + np.uint32(_KS[(i + 2) % 3] + np.uint32(i + 1))
    return x0 ^ x1


def _neg_log_unit(bits):
    fb = (bits >> 9) | np.uint32(0x3F800000)
    f = lax.bitcast_convert_type(fb, jnp.float32) - np.float32(1.0)
    span = np.float32(np.float32(1.0) - np.float32(1e-10))
    u = jnp.maximum(np.float32(1e-10), f * span + np.float32(1e-10))
    return -jnp.log(u)


def _body(p_ref, o_ref, *, blk_rows, n_elems):
    g = pl.program_id(0)
    p = p_ref[...]
    row = lax.broadcasted_iota(jnp.int32, (blk_rows, _LANES), 0)
    col = lax.broadcasted_iota(jnp.int32, (blk_rows, _LANES), 1)
    base = g * (blk_rows * _LANES)
    idx = (base + row * _LANES + col).astype(jnp.uint32)
    e0 = _neg_log_unit(_threefry_bits(idx))
    e1 = _neg_log_unit(_threefry_bits(idx + np.uint32(n_elems)))
    cond = (np.float32(1.0) - p) * e0 > p * e1
    o_ref[...] = cond.astype(jnp.float32)


def kernel(not_infected_probs):
    n = not_infected_probs.shape[0]
    pad = (-n) % _LANES
    p = not_infected_probs
    if pad:
        p = jnp.pad(p, (0, pad))
    rows = p.shape[0] // _LANES
    p2 = p.reshape(rows, _LANES)

    blk_rows = 512
    grid = (rows + blk_rows - 1) // blk_rows
    out = pl.pallas_call(
        functools.partial(_body, blk_rows=blk_rows, n_elems=n),
        grid=(grid,),
        in_specs=[pl.BlockSpec((blk_rows, _LANES), lambda g: (g, 0))],
        out_specs=pl.BlockSpec((blk_rows, _LANES), lambda g: (g, 0)),
        out_shape=jax.ShapeDtypeStruct((rows, _LANES), jnp.float32),
        compiler_params=pltpu.CompilerParams(
            dimension_semantics=("parallel",),
        ),
    )(p2)
    out = out.reshape(rows * _LANES)
    if pad:
        out = out[:n]
    return out
